# baseline (speedup 1.0000x reference)
import jax
import jax.numpy as jnp
from jax import lax
from jax.experimental import pallas as pl
from jax.experimental.pallas import tpu as pltpu

_C = 96
_H = 128
_W = 128
_HW = _H * _W
_PAD = _W
_NCLS = 15
_NBOX = 60


def _body(x_hbm, wc_ref, bc_ref, wh_ref, bh_ref,
          cls_ref, bbox_ref, s0, s1, xbuf, sem):
    i = pl.program_id(0)
    n = pl.num_programs(0)
    nslot = lax.rem(i + 1, 2)

    hrows = _H // 2
    _HALF = hrows * _W
    _EXT = (hrows + 1) * _W

    def _build(img_slot):
        col = lax.broadcasted_iota(jnp.int32, (_C, _EXT), 1) & (_W - 1)
        for ci, s in ((0, s0), (1, s1)):
            r0 = 0 if ci == 0 else hrows - 1
            dst = _PAD if ci == 0 else 0
            xc = xbuf[img_slot, :, r0:r0 + hrows + 1, :]
            xc = xc.astype(jnp.bfloat16).reshape(_C, _EXT)
            xl = jnp.where(col == 0, jnp.bfloat16(0),
                           jnp.roll(xc, 1, axis=1))
            xr = jnp.where(col == _W - 1, jnp.bfloat16(0),
                           jnp.roll(xc, -1, axis=1))
            s[0:_C, dst:dst + _EXT] = xc
            s[_C:2 * _C, dst:dst + _EXT] = xl
            s[2 * _C:3 * _C, dst:dst + _EXT] = xr

    @pl.when(i == 0)
    def _prologue():
        zpad = jnp.zeros((3 * _C, _PAD), jnp.bfloat16)
        s0[:, 0:_PAD] = zpad
        s1[:, _EXT:] = zpad
        pltpu.make_async_copy(x_hbm.at[0], xbuf.at[0], sem.at[0]).start()
        pltpu.make_async_copy(x_hbm.at[0], xbuf.at[0], sem.at[0]).wait()
        _build(0)

    @pl.when(i + 1 < n)
    def _prefetch():
        pltpu.make_async_copy(x_hbm.at[i + 1], xbuf.at[nslot],
                              sem.at[nslot]).start()

    for ci, s in ((0, s0), (1, s1)):
        acc = None
        for dy in (-1, 0, 1):
            off = _PAD + dy * _W
            part = lax.dot_general(
                wc_ref[dy + 1], s[:, off:off + _HALF],
                (((1,), (0,)), ((), ())),
                preferred_element_type=jnp.float32)
            acc = part if acc is None else acc + part
        h = jnp.maximum(acc + bc_ref[...], 0.0)
        hb = h.astype(jnp.bfloat16)

        y = lax.dot_general(wh_ref[...], hb, (((1,), (0,)), ((), ())),
                            preferred_element_type=jnp.float32) + bh_ref[...]
        cls_ref[0, :, ci * hrows:(ci + 1) * hrows] = (
            y[0:_NCLS].reshape(_NCLS, hrows, _W))
        bbox_ref[0, :, ci * hrows:(ci + 1) * hrows] = (
            y[16:16 + _NBOX].reshape(_NBOX, hrows, _W))

    @pl.when(i + 1 < n)
    def _build_next():
        pltpu.make_async_copy(x_hbm.at[i + 1], xbuf.at[nslot],
                              sem.at[nslot]).wait()
        _build(nslot)


def kernel(x, W_conv, b_conv, W_cls, b_cls, W_bbox, b_bbox):
    n = x.shape[0]
    wt = jnp.transpose(W_conv, (2, 3, 0, 1))
    wc = jnp.concatenate([wt[:, 1], wt[:, 0], wt[:, 2]], axis=-1)
    wc = wc.astype(jnp.bfloat16)
    wh = jnp.zeros((80, _C), jnp.float32)
    wh = wh.at[0:_NCLS].set(W_cls.reshape(_NCLS, _C))
    wh = wh.at[16:16 + _NBOX].set(W_bbox.reshape(_NBOX, _C))
    wh = wh.astype(jnp.bfloat16)
    bc = b_conv.reshape(_C, 1)
    bh = jnp.zeros((80, 1), jnp.float32)
    bh = bh.at[0:_NCLS].set(b_cls.reshape(_NCLS, 1))
    bh = bh.at[16:16 + _NBOX].set(b_bbox.reshape(_NBOX, 1))

    logits, bbox = pl.pallas_call(
        _body,
        grid=(n,),
        in_specs=[
            pl.BlockSpec(memory_space=pl.ANY),
            pl.BlockSpec((3, _C, 3 * _C), lambda i: (0, 0, 0)),
            pl.BlockSpec((_C, 1), lambda i: (0, 0)),
            pl.BlockSpec((80, _C), lambda i: (0, 0)),
            pl.BlockSpec((80, 1), lambda i: (0, 0)),
        ],
        out_specs=[
            pl.BlockSpec((1, _NCLS, _H, _W), lambda i: (i, 0, 0, 0)),
            pl.BlockSpec((1, _NBOX, _H, _W), lambda i: (i, 0, 0, 0)),
        ],
        out_shape=[
            jax.ShapeDtypeStruct((n, _NCLS, _H, _W), jnp.float32),
            jax.ShapeDtypeStruct((n, _NBOX, _H, _W), jnp.float32),
        ],
        scratch_shapes=[
            pltpu.VMEM((3 * _C, _PAD + (_H // 2 + 1) * _W), jnp.bfloat16),
            pltpu.VMEM((3 * _C, _PAD + (_H // 2 + 1) * _W), jnp.bfloat16),
            pltpu.VMEM((2, _C, _H, _W), jnp.float32),
            pltpu.SemaphoreType.DMA((2,)),
        ],
    )(x, wc, bc, wh, bh)

    return (logits, bbox)


# speedup vs baseline: 1.0570x; 1.0570x over previous
import jax
import jax.numpy as jnp
from jax import lax
from jax.experimental import pallas as pl
from jax.experimental.pallas import tpu as pltpu

_C = 96
_H = 128
_W = 128
_HW = _H * _W
_PAD = _W
_NCLS = 15
_NBOX = 60
_HR = _H // 2
_HALF = _HR * _W
_EXT = (_HR + 1) * _W


def _body(x_hbm, wc_ref, bc_ref, wh_ref, bh_ref,
          cls_ref, bbox_ref, s0a, s1a, s0b, s1b, xbuf, sem):
    j = pl.program_id(0)
    nj = pl.num_programs(0)

    def _build(img_slot, s0, s1):
        col = lax.broadcasted_iota(jnp.int32, (_C, _EXT), 1) & (_W - 1)
        for ci, s in ((0, s0), (1, s1)):
            r0 = 0 if ci == 0 else _HR - 1
            dst = _PAD if ci == 0 else 0
            xc = xbuf[img_slot, :, r0:r0 + _HR + 1, :]
            xc = xc.astype(jnp.bfloat16).reshape(_C, _EXT)
            xl = jnp.where(col == 0, jnp.bfloat16(0),
                           jnp.roll(xc, 1, axis=1))
            xr = jnp.where(col == _W - 1, jnp.bfloat16(0),
                           jnp.roll(xc, -1, axis=1))
            s[0:_C, dst:dst + _EXT] = xc
            s[_C:2 * _C, dst:dst + _EXT] = xl
            s[2 * _C:3 * _C, dst:dst + _EXT] = xr

    def _compute(s0, s1, oi):
        for ci, s in ((0, s0), (1, s1)):
            acc = None
            for dy in (-1, 0, 1):
                off = _PAD + dy * _W
                part = lax.dot_general(
                    wc_ref[dy + 1], s[:, off:off + _HALF],
                    (((1,), (0,)), ((), ())),
                    preferred_element_type=jnp.float32)
                acc = part if acc is None else acc + part
            h = jnp.maximum(acc + bc_ref[...], 0.0)
            hb = h.astype(jnp.bfloat16)

            y = lax.dot_general(wh_ref[...], hb, (((1,), (0,)), ((), ())),
                                preferred_element_type=jnp.float32)
            y = y + bh_ref[...]
            cls_ref[oi, :, ci * _HR:(ci + 1) * _HR] = (
                y[0:_NCLS].reshape(_NCLS, _HR, _W))
            bbox_ref[oi, :, ci * _HR:(ci + 1) * _HR] = (
                y[16:16 + _NBOX].reshape(_NBOX, _HR, _W))

    @pl.when(j == 0)
    def _prologue():
        zpad = jnp.zeros((3 * _C, _PAD), jnp.bfloat16)
        for s0, s1 in ((s0a, s1a), (s0b, s1b)):
            s0[:, 0:_PAD] = zpad
            s1[:, _EXT:] = zpad
        pltpu.make_async_copy(x_hbm.at[0], xbuf.at[0], sem.at[0]).start()
        pltpu.make_async_copy(x_hbm.at[0], xbuf.at[0], sem.at[0]).wait()
        _build(0, s0a, s1a)

    pltpu.make_async_copy(x_hbm.at[2 * j + 1], xbuf.at[1],
                          sem.at[1]).start()

    @pl.when(j + 1 < nj)
    def _prefetch_even():
        pltpu.make_async_copy(x_hbm.at[2 * j + 2], xbuf.at[0],
                              sem.at[0]).start()

    _compute(s0a, s1a, 0)

    pltpu.make_async_copy(x_hbm.at[2 * j + 1], xbuf.at[1],
                          sem.at[1]).wait()
    _build(1, s0b, s1b)

    _compute(s0b, s1b, 1)

    @pl.when(j + 1 < nj)
    def _build_next():
        pltpu.make_async_copy(x_hbm.at[2 * j + 2], xbuf.at[0],
                              sem.at[0]).wait()
        _build(0, s0a, s1a)


def kernel(x, W_conv, b_conv, W_cls, b_cls, W_bbox, b_bbox):
    n = x.shape[0]
    wt = jnp.transpose(W_conv, (2, 3, 0, 1))
    wc = jnp.concatenate([wt[:, 1], wt[:, 0], wt[:, 2]], axis=-1)
    wc = wc.astype(jnp.bfloat16)
    wh = jnp.zeros((80, _C), jnp.float32)
    wh = wh.at[0:_NCLS].set(W_cls.reshape(_NCLS, _C))
    wh = wh.at[16:16 + _NBOX].set(W_bbox.reshape(_NBOX, _C))
    wh = wh.astype(jnp.bfloat16)
    bc = b_conv.reshape(_C, 1)
    bh = jnp.zeros((80, 1), jnp.float32)
    bh = bh.at[0:_NCLS].set(b_cls.reshape(_NCLS, 1))
    bh = bh.at[16:16 + _NBOX].set(b_bbox.reshape(_NBOX, 1))

    logits, bbox = pl.pallas_call(
        _body,
        grid=(n // 2,),
        in_specs=[
            pl.BlockSpec(memory_space=pl.ANY),
            pl.BlockSpec((3, _C, 3 * _C), lambda j: (0, 0, 0)),
            pl.BlockSpec((_C, 1), lambda j: (0, 0)),
            pl.BlockSpec((80, _C), lambda j: (0, 0)),
            pl.BlockSpec((80, 1), lambda j: (0, 0)),
        ],
        out_specs=[
            pl.BlockSpec((2, _NCLS, _H, _W), lambda j: (j, 0, 0, 0)),
            pl.BlockSpec((2, _NBOX, _H, _W), lambda j: (j, 0, 0, 0)),
        ],
        out_shape=[
            jax.ShapeDtypeStruct((n, _NCLS, _H, _W), jnp.float32),
            jax.ShapeDtypeStruct((n, _NBOX, _H, _W), jnp.float32),
        ],
        scratch_shapes=[
            pltpu.VMEM((3 * _C, _PAD + _EXT), jnp.bfloat16),
            pltpu.VMEM((3 * _C, _PAD + _EXT), jnp.bfloat16),
            pltpu.VMEM((3 * _C, _PAD + _EXT), jnp.bfloat16),
            pltpu.VMEM((3 * _C, _PAD + _EXT), jnp.bfloat16),
            pltpu.VMEM((2, _C, _H, _W), jnp.float32),
            pltpu.SemaphoreType.DMA((2,)),
        ],
    )(x, wc, bc, wh, bh)

    return (logits, bbox)


# speedup vs baseline: 1.1094x; 1.0495x over previous
import jax
import jax.numpy as jnp
from jax import lax
from jax.experimental import pallas as pl
from jax.experimental.pallas import tpu as pltpu

_C = 96
_H = 128
_W = 128
_HW = _H * _W
_PAD = _W
_NCLS = 15
_NBOX = 60


def _body(x_hbm, wc_ref, bc_ref, wh_ref, bh_ref,
          cls_ref, bbox_ref, s, xbuf, sem):
    i = pl.program_id(0)
    n = pl.num_programs(0)
    slot = lax.rem(i, 2)
    nslot = lax.rem(i + 1, 2)

    @pl.when(i == 0)
    def _prologue():
        pltpu.make_async_copy(x_hbm.at[0], xbuf.at[0], sem.at[0]).start()

    @pl.when(i + 1 < n)
    def _prefetch():
        pltpu.make_async_copy(x_hbm.at[i + 1], xbuf.at[nslot],
                              sem.at[nslot]).start()

    pltpu.make_async_copy(x_hbm.at[i], xbuf.at[slot], sem.at[slot]).wait()

    @pl.when(i == 0)
    def _zero_pads():
        zpad = jnp.zeros((3 * _C, _PAD), jnp.bfloat16)
        s[:, 0:_PAD] = zpad
        s[:, _PAD + _HW:] = zpad

    xb = xbuf[slot].astype(jnp.bfloat16).reshape(_C, _HW)

    s[0:_C, _PAD:_PAD + _HW] = xb
    col = lax.broadcasted_iota(jnp.int32, (_C, _HW), 1) & (_W - 1)
    xl = jnp.where(col == 0, jnp.bfloat16(0), jnp.roll(xb, 1, axis=1))
    s[_C:2 * _C, _PAD:_PAD + _HW] = xl
    xr = jnp.where(col == _W - 1, jnp.bfloat16(0), jnp.roll(xb, -1, axis=1))
    s[2 * _C:3 * _C, _PAD:_PAD + _HW] = xr

    acc = None
    for dy in (-1, 0, 1):
        off = _PAD + dy * _W
        part = lax.dot_general(
            wc_ref[dy + 1], s[:, off:off + _HW], (((1,), (0,)), ((), ())),
            preferred_element_type=jnp.float32)
        acc = part if acc is None else acc + part
    h = jnp.maximum(acc + bc_ref[...], 0.0)
    hb = h.astype(jnp.bfloat16)

    y = lax.dot_general(wh_ref[...], hb, (((1,), (0,)), ((), ())),
                        preferred_element_type=jnp.float32) + bh_ref[...]
    cls_ref[0] = y[0:_NCLS].reshape(_NCLS, _H, _W)
    bbox_ref[0] = y[16:16 + _NBOX].reshape(_NBOX, _H, _W)


def kernel(x, W_conv, b_conv, W_cls, b_cls, W_bbox, b_bbox):
    n = x.shape[0]
    wt = jnp.transpose(W_conv, (2, 3, 0, 1))
    wc = jnp.concatenate([wt[:, 1], wt[:, 0], wt[:, 2]], axis=-1)
    wc = wc.astype(jnp.bfloat16)
    wh = jnp.zeros((80, _C), jnp.float32)
    wh = wh.at[0:_NCLS].set(W_cls.reshape(_NCLS, _C))
    wh = wh.at[16:16 + _NBOX].set(W_bbox.reshape(_NBOX, _C))
    wh = wh.astype(jnp.bfloat16)
    bc = b_conv.reshape(_C, 1)
    bh = jnp.zeros((80, 1), jnp.float32)
    bh = bh.at[0:_NCLS].set(b_cls.reshape(_NCLS, 1))
    bh = bh.at[16:16 + _NBOX].set(b_bbox.reshape(_NBOX, 1))

    logits, bbox = pl.pallas_call(
        _body,
        grid=(n,),
        in_specs=[
            pl.BlockSpec(memory_space=pl.ANY),
            pl.BlockSpec((3, _C, 3 * _C), lambda i: (0, 0, 0)),
            pl.BlockSpec((_C, 1), lambda i: (0, 0)),
            pl.BlockSpec((80, _C), lambda i: (0, 0)),
            pl.BlockSpec((80, 1), lambda i: (0, 0)),
        ],
        out_specs=[
            pl.BlockSpec((1, _NCLS, _H, _W), lambda i: (i, 0, 0, 0)),
            pl.BlockSpec((1, _NBOX, _H, _W), lambda i: (i, 0, 0, 0)),
        ],
        out_shape=[
            jax.ShapeDtypeStruct((n, _NCLS, _H, _W), jnp.float32),
            jax.ShapeDtypeStruct((n, _NBOX, _H, _W), jnp.float32),
        ],
        scratch_shapes=[
            pltpu.VMEM((3 * _C, _HW + 2 * _PAD), jnp.bfloat16),
            pltpu.VMEM((2, _C, _H, _W), jnp.float32),
            pltpu.SemaphoreType.DMA((2,)),
        ],
    )(x, wc, bc, wh, bh)

    return (logits, bbox)
